# baseline (speedup 1.0000x reference)
import functools

import jax
import jax.numpy as jnp
from jax import lax
from jax.experimental import pallas as pl
from jax.experimental.pallas import tpu as pltpu
from jax.experimental.pallas import tpu_sc as plsc

N = 10000
E = 160000
IN = 128
ED = 16
H = 8
NG = 100

_NC = 2
_NS = 16
_NW = _NC * _NS
_CH = 128
_NCHK = E // _CH
_GI = 39
_SI = 78
_NP = 10240
_NPH = _NP // 2
_TR = _NPH + 128
_RPC = _TR // _NS


def _sc_mesh():
    return plsc.VectorSubcoreMesh(core_axis_name="c", subcore_axis_name="s")


def _sc_gather():
    @functools.partial(
        pl.kernel,
        mesh=_sc_mesh(),
        out_type=jax.ShapeDtypeStruct((E, 128), jnp.float32),
        scratch_types=[
            pltpu.VMEM((_CH,), jnp.int32),
            pltpu.VMEM((_CH,), jnp.int32),
            pltpu.VMEM((_CH, 128), jnp.float32),
            pltpu.VMEM((_CH, 128), jnp.float32),
            pltpu.SemaphoreType.DMA,
            pltpu.SemaphoreType.DMA,
        ],
    )
    def g(idx_hbm, table_hbm, out_hbm, idxA, idxB, rowA, rowB, semA, semB):
        w = lax.axis_index("s") * _NC + lax.axis_index("c")

        def fetch(c, idxv, rowv, sem):
            pltpu.sync_copy(idx_hbm.at[pl.ds(pl.multiple_of(c * _CH, 8), _CH)],
                            idxv)
            pltpu.async_copy(table_hbm.at[idxv], rowv, sem)

        def drain(idxv, rowv, sem):
            pltpu.make_async_copy(table_hbm.at[idxv], rowv, sem).wait()

        def store(c, rowv):
            pltpu.sync_copy(
                rowv, out_hbm.at[pl.ds(pl.multiple_of(c * _CH, 8), _CH)])

        fetch(w, idxA, rowA, semA)

        def pair(j, carry):
            cA = w + 32 * (2 * j)
            fetch(cA + 32, idxB, rowB, semB)
            drain(idxA, rowA, semA)
            store(cA, rowA)

            @pl.when(2 * j + 2 < _GI)
            def _():
                fetch(cA + 64, idxA, rowA, semA)

            drain(idxB, rowB, semB)
            store(cA + 32, rowB)
            return carry

        lax.fori_loop(0, _GI // 2, pair, 0)
        cL = w + 32 * (_GI - 1)
        drain(idxA, rowA, semA)
        store(cL, rowA)

        @pl.when(w < 2)
        def _():
            cT = _NW * _GI + w
            fetch(cT, idxA, rowA, semA)
            drain(idxA, rowA, semA)
            store(cT, rowA)

    return g


def _sc_scatter_add():
    @functools.partial(
        pl.kernel,
        mesh=_sc_mesh(),
        out_type=jax.ShapeDtypeStruct((2 * _TR, 128), jnp.float32),
        scratch_types=[
            pltpu.VMEM((_CH,), jnp.int32),
            pltpu.VMEM((_CH,), jnp.int32),
            pltpu.VMEM((_CH, 128), jnp.float32),
            pltpu.VMEM((_CH, 128), jnp.float32),
            pltpu.VMEM((_RPC, 128), jnp.float32),
            pltpu.VMEM_SHARED((_TR, 128), jnp.float32),
            pltpu.SemaphoreType.DMA,
            pltpu.SemaphoreType.DMA,
        ],
    )
    def s(idx_hbm, msg_hbm, out_hbm, idxA, idxB, msgA, msgB, tmp_v, acc,
          semA, semB):
        cid = lax.axis_index("c")
        sid = lax.axis_index("s")
        lo = cid * _NPH

        def zbody(i, carry):
            for j in range(8):
                tmp_v[i, pl.ds(j * 16, 16)] = jnp.zeros((16,), jnp.float32)
            return carry

        lax.fori_loop(0, _RPC, zbody, 0)
        pltpu.sync_copy(tmp_v, acc.at[pl.ds(sid * _RPC, _RPC)])
        plsc.subcore_barrier()

        def fetch(c, idxv, msgv, sem):
            pltpu.sync_copy(idx_hbm.at[pl.ds(pl.multiple_of(c * _CH, 8), _CH)],
                            idxv)
            pltpu.async_copy(
                msg_hbm.at[pl.ds(pl.multiple_of(c * _CH, 8), _CH)], msgv, sem)
            for k in range(8):
                v = idxv[pl.ds(k * 16, 16)] - lo
                ok = (v >= 0) & (v < _NPH)
                idxv[pl.ds(k * 16, 16)] = jnp.where(ok, v, _NPH)

        def accum(c, idxv, msgv, sem):
            pltpu.make_async_copy(
                msg_hbm.at[pl.ds(pl.multiple_of(c * _CH, 8), _CH)], msgv,
                sem).wait()
            pltpu.sync_copy(msgv, acc.at[idxv], add=True)

        fetch(sid, idxA, msgA, semA)

        def pair(j, carry):
            cA = sid + 16 * (2 * j)
            fetch(cA + 16, idxB, msgB, semB)
            accum(cA, idxA, msgA, semA)

            @pl.when(2 * j + 2 < _SI)
            def _():
                fetch(cA + 32, idxA, msgA, semA)

            accum(cA + 16, idxB, msgB, semB)
            return carry

        lax.fori_loop(0, _SI // 2, pair, 0)

        @pl.when(sid < 2)
        def _():
            cT = _NS * _SI + sid
            fetch(cT, idxA, msgA, semA)
            accum(cT, idxA, msgA, semA)

        plsc.subcore_barrier()
        pltpu.sync_copy(acc.at[pl.ds(sid * _RPC, _RPC)], tmp_v)
        pltpu.sync_copy(
            tmp_v,
            out_hbm.at[pl.ds(pl.multiple_of(cid * _TR + sid * _RPC, 8), _RPC)])

    return s


_B1 = 640
_B3 = 2000


def _bdot(a, b):
    return jnp.dot(a.astype(jnp.bfloat16), b,
                   preferred_element_type=jnp.float32)


def _k1_body(ea_ref, xs_ref, Wa_ref, ba_ref, Wp_ref, bp_ref, sel_ref, c8_ref,
             out_ref):
    eh = jnp.maximum(ea_ref[...] @ Wa_ref[...] + ba_ref[...], 0.0)
    W = _bdot(eh, Wp_ref[...]) + bp_ref[...]
    xst = jnp.concatenate([xs_ref[...]] * H, axis=1)
    out_ref[...] = _bdot(W * xst, sel_ref[...]) + c8_ref[...]


def _k3_body(ea_ref, hs_ref, Wa_ref, ba_ref, Wp_ref, bp_ref, sel_ref, out_ref):
    eh = jnp.maximum(ea_ref[...] @ Wa_ref[...] + ba_ref[...], 0.0)
    W = _bdot(eh, Wp_ref[...]) + bp_ref[...]
    out_ref[...] = _bdot(W * hs_ref[...], sel_ref[...])


def _seg(s_ref):
    return jnp.concatenate(
        [s_ref[:_NPH], s_ref[_TR:_TR + (N - _NPH)]], axis=0)


def _k2_body(s1p_ref, x_ref, Wr_ref, b_ref, g_ref, bt_ref, rep_ref,
             h1t_ref, cnt_ref):
    s = _seg(s1p_ref)
    cnt = jnp.maximum(s[:, 8:9], 1.0)
    pre = s[:, :H] / cnt + x_ref[...] @ Wr_ref[...] + b_ref[...]
    m = jnp.mean(pre, axis=0, keepdims=True)
    v = jnp.mean((pre - m) ** 2, axis=0, keepdims=True)
    h = jnp.maximum((pre - m) / jnp.sqrt(v + 1e-5) * g_ref[...] + bt_ref[...],
                    0.0)
    h1t_ref[...] = h @ rep_ref[...]
    cnt_ref[...] = jnp.broadcast_to(cnt, (N, 8))


_NGP = 104


def _k4_body(s2p_ref, h1t_ref, cnt_ref, batch_ref, Wr_ref, b_ref, g_ref,
             bt_ref, Wf1_ref, bf1_ref, Wf2_ref, bf2_ref, Wf3_ref, bf3_ref,
             out_ref):
    s = _seg(s2p_ref)
    cnt = jnp.maximum(cnt_ref[:, :1], 1.0)
    h1 = h1t_ref[:, :H]
    pre = s[:, :H] / cnt + h1 @ Wr_ref[...] + b_ref[...]
    m = jnp.mean(pre, axis=0, keepdims=True)
    v = jnp.mean((pre - m) ** 2, axis=0, keepdims=True)
    h2 = jnp.maximum((pre - m) / jnp.sqrt(v + 1e-5) * g_ref[...] + bt_ref[...],
                     0.0)

    row = lax.broadcasted_iota(jnp.int32, (_NGP, N), 0)
    ohT = (row == batch_ref[...]).astype(jnp.float32)
    gs = jnp.dot(ohT, h2)
    cg = jnp.maximum(jnp.sum(ohT, axis=1, keepdims=True), 1.0)
    z = jnp.concatenate([gs / cg, gs], axis=1)
    z = jnp.maximum(z @ Wf1_ref[...] + bf1_ref[...], 0.0)
    z = jnp.maximum(z @ Wf2_ref[...] + bf2_ref[...], 0.0)
    out_ref[...] = (z @ Wf3_ref[...] + bf3_ref[...])[:NG]


def _full(shape):
    return pl.BlockSpec(shape, lambda: tuple(0 for _ in shape))


def kernel(x, edge_index, edge_attr, batch,
           We1a, be1a, We1b, be1b, Wr1, b1, g1, bt1,
           We2a, be2a, We2b, be2b, Wr2, b2, g2, bt2,
           Wf1, bf1, Wf2, bf2, Wf3, bf3):
    src = edge_index[0]
    dst = edge_index[1]

    W1p = We1b.reshape(128, IN, H).transpose(0, 2, 1).reshape(128, H * IN)
    b1p = be1b.reshape(IN, H).transpose(1, 0).reshape(1, H * IN)
    W2p = jnp.zeros((256, 128), jnp.float32)
    W2p = W2p.at[:, :H * H].set(
        We2b.reshape(256, H, H).transpose(0, 2, 1).reshape(256, H * H))
    b2p = jnp.zeros((1, 128), jnp.float32)
    b2p = b2p.at[:, :H * H].set(be2b.reshape(H, H).transpose(1, 0).reshape(1, H * H))
    j1 = jnp.arange(H * IN)[:, None]
    sel1 = ((j1 // IN == jnp.arange(128)[None, :])
            & (j1 // IN < H)).astype(jnp.float32)
    c8 = (jnp.arange(128)[None, :] == 8).astype(jnp.float32)
    j2 = jnp.arange(128)[:, None]
    sel2 = ((j2 // H == jnp.arange(128)[None, :])
            & (j2 < H * H)).astype(jnp.float32)
    ji = jnp.arange(128)[None, :]
    rep = ((jnp.arange(H)[:, None] == ji % H)
           & (ji < H * H)).astype(jnp.float32)

    gather = _sc_gather()
    scat = _sc_scatter_add()

    xs = gather(src, x)

    msg1 = pl.pallas_call(
        _k1_body,
        grid=(E // _B1,),
        in_specs=[
            pl.BlockSpec((_B1, ED), lambda i: (i, 0)),
            pl.BlockSpec((_B1, IN), lambda i: (i, 0)),
            pl.BlockSpec((ED, 128), lambda i: (0, 0)),
            pl.BlockSpec((1, 128), lambda i: (0, 0)),
            pl.BlockSpec((128, H * IN), lambda i: (0, 0)),
            pl.BlockSpec((1, H * IN), lambda i: (0, 0)),
            pl.BlockSpec((H * IN, 128), lambda i: (0, 0)),
            pl.BlockSpec((1, 128), lambda i: (0, 0)),
        ],
        out_specs=pl.BlockSpec((_B1, 128), lambda i: (i, 0)),
        out_shape=jax.ShapeDtypeStruct((E, 128), jnp.float32),
    )(edge_attr, xs, We1a, be1a.reshape(1, 128),
      W1p.astype(jnp.bfloat16), b1p, sel1.astype(jnp.bfloat16), c8)

    s1p = scat(dst, msg1)

    h1t, cnt8 = pl.pallas_call(
        _k2_body,
        in_specs=[_full((2 * _TR, 128)), _full((N, IN)), _full((IN, H)),
                  _full((1, H)), _full((1, H)), _full((1, H)), _full((H, 128))],
        out_specs=[_full((N, 128)), _full((N, 8))],
        out_shape=[jax.ShapeDtypeStruct((N, 128), jnp.float32),
                   jax.ShapeDtypeStruct((N, 8), jnp.float32)],
    )(s1p, x, Wr1, b1.reshape(1, H), g1.reshape(1, H), bt1.reshape(1, H), rep)

    hs = gather(src, h1t)

    msg2 = pl.pallas_call(
        _k3_body,
        grid=(E // _B3,),
        in_specs=[
            pl.BlockSpec((_B3, ED), lambda i: (i, 0)),
            pl.BlockSpec((_B3, 128), lambda i: (i, 0)),
            pl.BlockSpec((ED, 256), lambda i: (0, 0)),
            pl.BlockSpec((1, 256), lambda i: (0, 0)),
            pl.BlockSpec((256, 128), lambda i: (0, 0)),
            pl.BlockSpec((1, 128), lambda i: (0, 0)),
            pl.BlockSpec((128, 128), lambda i: (0, 0)),
        ],
        out_specs=pl.BlockSpec((_B3, 128), lambda i: (i, 0)),
        out_shape=jax.ShapeDtypeStruct((E, 128), jnp.float32),
    )(edge_attr, hs, We2a, be2a.reshape(1, 256),
      W2p.astype(jnp.bfloat16), b2p, sel2.astype(jnp.bfloat16))

    s2p = scat(dst, msg2)

    out = pl.pallas_call(
        _k4_body,
        in_specs=[_full((2 * _TR, 128)), _full((N, 128)), _full((N, 8)),
                  _full((1, N)), _full((H, H)), _full((1, H)), _full((1, H)),
                  _full((1, H)), _full((2 * H, H)), _full((1, H)),
                  _full((H, H // 2)), _full((1, H // 2)),
                  _full((H // 2, 1)), _full((1, 1))],
        out_specs=_full((NG, 1)),
        out_shape=jax.ShapeDtypeStruct((NG, 1), jnp.float32),
    )(s2p, h1t, cnt8, batch.reshape(1, N), Wr2, b2.reshape(1, H),
      g2.reshape(1, H), bt2.reshape(1, H), Wf1, bf1.reshape(1, H),
      Wf2, bf2.reshape(1, H // 2), Wf3, bf3.reshape(1, 1))

    return out


# speedup vs baseline: 1.1219x; 1.1219x over previous
import functools

import jax
import jax.numpy as jnp
from jax import lax
from jax.experimental import pallas as pl
from jax.experimental.pallas import tpu as pltpu
from jax.experimental.pallas import tpu_sc as plsc

N = 10000
E = 160000
IN = 128
ED = 16
H = 8
NG = 100

_NC = 2
_NS = 16
_NW = _NC * _NS
_CH = 128
_NCHK = E // _CH
_GI = 39
_SI = 78
_NP = 10240
_NPH = _NP // 2
_TR = _NPH + 128
_RPC = _TR // _NS


def _sc_mesh():
    return plsc.VectorSubcoreMesh(core_axis_name="c", subcore_axis_name="s")


def _sc_gather(dtype):
    @functools.partial(
        pl.kernel,
        mesh=_sc_mesh(),
        out_type=jax.ShapeDtypeStruct((E, 128), dtype),
        scratch_types=[
            pltpu.VMEM((_CH,), jnp.int32),
            pltpu.VMEM((_CH,), jnp.int32),
            pltpu.VMEM((_CH, 128), dtype),
            pltpu.VMEM((_CH, 128), dtype),
            pltpu.SemaphoreType.DMA,
            pltpu.SemaphoreType.DMA,
        ],
    )
    def g(idx_hbm, table_hbm, out_hbm, idxA, idxB, rowA, rowB, semA, semB):
        w = lax.axis_index("s") * _NC + lax.axis_index("c")

        def fetch(c, idxv, rowv, sem):
            pltpu.sync_copy(idx_hbm.at[pl.ds(pl.multiple_of(c * _CH, 8), _CH)],
                            idxv)
            pltpu.async_copy(table_hbm.at[idxv], rowv, sem)

        def drain(idxv, rowv, sem):
            pltpu.make_async_copy(table_hbm.at[idxv], rowv, sem).wait()

        def store(c, rowv):
            pltpu.sync_copy(
                rowv, out_hbm.at[pl.ds(pl.multiple_of(c * _CH, 8), _CH)])

        fetch(w, idxA, rowA, semA)

        def pair(j, carry):
            cA = w + 32 * (2 * j)
            fetch(cA + 32, idxB, rowB, semB)
            drain(idxA, rowA, semA)
            store(cA, rowA)

            @pl.when(2 * j + 2 < _GI)
            def _():
                fetch(cA + 64, idxA, rowA, semA)

            drain(idxB, rowB, semB)
            store(cA + 32, rowB)
            return carry

        lax.fori_loop(0, _GI // 2, pair, 0)
        cL = w + 32 * (_GI - 1)
        drain(idxA, rowA, semA)
        store(cL, rowA)

        @pl.when(w < 2)
        def _():
            cT = _NW * _GI + w
            fetch(cT, idxA, rowA, semA)
            drain(idxA, rowA, semA)
            store(cT, rowA)

    return g


def _sc_scatter_add():
    @functools.partial(
        pl.kernel,
        mesh=_sc_mesh(),
        out_type=jax.ShapeDtypeStruct((2 * _TR, 128), jnp.float32),
        scratch_types=[
            pltpu.VMEM((_CH,), jnp.int32),
            pltpu.VMEM((_CH,), jnp.int32),
            pltpu.VMEM((_CH, 128), jnp.float32),
            pltpu.VMEM((_CH, 128), jnp.float32),
            pltpu.VMEM((_RPC, 128), jnp.float32),
            pltpu.VMEM_SHARED((_TR, 128), jnp.float32),
            pltpu.SemaphoreType.DMA,
            pltpu.SemaphoreType.DMA,
        ],
    )
    def s(idx_hbm, msg_hbm, out_hbm, idxA, idxB, msgA, msgB, tmp_v, acc,
          semA, semB):
        cid = lax.axis_index("c")
        sid = lax.axis_index("s")
        lo = cid * _NPH

        def zbody(i, carry):
            for j in range(8):
                tmp_v[i, pl.ds(j * 16, 16)] = jnp.zeros((16,), jnp.float32)
            return carry

        lax.fori_loop(0, _RPC, zbody, 0)
        pltpu.sync_copy(tmp_v, acc.at[pl.ds(sid * _RPC, _RPC)])
        plsc.subcore_barrier()

        def fetch(c, idxv, msgv, sem):
            pltpu.sync_copy(idx_hbm.at[pl.ds(pl.multiple_of(c * _CH, 8), _CH)],
                            idxv)
            pltpu.async_copy(
                msg_hbm.at[pl.ds(pl.multiple_of(c * _CH, 8), _CH)], msgv, sem)
            for k in range(8):
                v = idxv[pl.ds(k * 16, 16)] - lo
                ok = (v >= 0) & (v < _NPH)
                idxv[pl.ds(k * 16, 16)] = jnp.where(ok, v, _NPH)

        def accum(c, idxv, msgv, sem):
            pltpu.make_async_copy(
                msg_hbm.at[pl.ds(pl.multiple_of(c * _CH, 8), _CH)], msgv,
                sem).wait()
            pltpu.sync_copy(msgv, acc.at[idxv], add=True)

        fetch(sid, idxA, msgA, semA)

        def pair(j, carry):
            cA = sid + 16 * (2 * j)
            fetch(cA + 16, idxB, msgB, semB)
            accum(cA, idxA, msgA, semA)

            @pl.when(2 * j + 2 < _SI)
            def _():
                fetch(cA + 32, idxA, msgA, semA)

            accum(cA + 16, idxB, msgB, semB)
            return carry

        lax.fori_loop(0, _SI // 2, pair, 0)

        @pl.when(sid < 2)
        def _():
            cT = _NS * _SI + sid
            fetch(cT, idxA, msgA, semA)
            accum(cT, idxA, msgA, semA)

        plsc.subcore_barrier()
        pltpu.sync_copy(acc.at[pl.ds(sid * _RPC, _RPC)], tmp_v)
        pltpu.sync_copy(
            tmp_v,
            out_hbm.at[pl.ds(pl.multiple_of(cid * _TR + sid * _RPC, 8), _RPC)])

    return s


_B1 = 1280
_B3 = 4000


def _bdot(a, b):
    return jnp.dot(a.astype(jnp.bfloat16), b,
                   preferred_element_type=jnp.float32)


def _k1_body(ea_ref, xs_ref, Wa_ref, ba_ref, Wp_ref, bp_ref, sel_ref, c8_ref,
             out_ref):
    eh = jnp.maximum(ea_ref[...] @ Wa_ref[...] + ba_ref[...], 0.0)
    W = _bdot(eh, Wp_ref[...]) + bp_ref[...]
    xst = jnp.concatenate([xs_ref[...]] * H, axis=1)
    out_ref[...] = _bdot(W * xst, sel_ref[...]) + c8_ref[...]


def _k3_body(ea_ref, hs_ref, Wa_ref, ba_ref, Wp_ref, bp_ref, sel_ref, out_ref):
    eh = jnp.maximum(ea_ref[...] @ Wa_ref[...] + ba_ref[...], 0.0)
    W = _bdot(eh, Wp_ref[...]) + bp_ref[...]
    out_ref[...] = _bdot(W * hs_ref[...], sel_ref[...])


def _seg(s_ref):
    return jnp.concatenate(
        [s_ref[:_NPH], s_ref[_TR:_TR + (N - _NPH)]], axis=0)


def _k2_body(s1p_ref, x_ref, Wr_ref, b_ref, g_ref, bt_ref, rep_ref,
             h1t_ref, cnt_ref):
    s = _seg(s1p_ref)
    cnt = jnp.maximum(s[:, 8:9], 1.0)
    pre = s[:, :H] / cnt + x_ref[...] @ Wr_ref[...] + b_ref[...]
    m = jnp.mean(pre, axis=0, keepdims=True)
    v = jnp.mean((pre - m) ** 2, axis=0, keepdims=True)
    h = jnp.maximum((pre - m) / jnp.sqrt(v + 1e-5) * g_ref[...] + bt_ref[...],
                    0.0)
    h1t_ref[...] = h @ rep_ref[...]
    cnt_ref[...] = jnp.broadcast_to(cnt, (N, 8))


_NGP = 104


def _k4_body(s2p_ref, h1t_ref, cnt_ref, batch_ref, Wr_ref, b_ref, g_ref,
             bt_ref, Wf1_ref, bf1_ref, Wf2_ref, bf2_ref, Wf3_ref, bf3_ref,
             out_ref):
    s = _seg(s2p_ref)
    cnt = jnp.maximum(cnt_ref[:, :1], 1.0)
    h1 = h1t_ref[:, :H]
    pre = s[:, :H] / cnt + h1 @ Wr_ref[...] + b_ref[...]
    m = jnp.mean(pre, axis=0, keepdims=True)
    v = jnp.mean((pre - m) ** 2, axis=0, keepdims=True)
    h2 = jnp.maximum((pre - m) / jnp.sqrt(v + 1e-5) * g_ref[...] + bt_ref[...],
                     0.0)

    row = lax.broadcasted_iota(jnp.int32, (_NGP, N), 0)
    ohT = (row == batch_ref[...]).astype(jnp.float32)
    gs = jnp.dot(ohT, h2)
    cg = jnp.maximum(jnp.sum(ohT, axis=1, keepdims=True), 1.0)
    z = jnp.concatenate([gs / cg, gs], axis=1)
    z = jnp.maximum(z @ Wf1_ref[...] + bf1_ref[...], 0.0)
    z = jnp.maximum(z @ Wf2_ref[...] + bf2_ref[...], 0.0)
    out_ref[...] = (z @ Wf3_ref[...] + bf3_ref[...])[:NG]


def _full(shape):
    return pl.BlockSpec(shape, lambda: tuple(0 for _ in shape))


def kernel(x, edge_index, edge_attr, batch,
           We1a, be1a, We1b, be1b, Wr1, b1, g1, bt1,
           We2a, be2a, We2b, be2b, Wr2, b2, g2, bt2,
           Wf1, bf1, Wf2, bf2, Wf3, bf3):
    src = edge_index[0]
    dst = edge_index[1]

    W1p = We1b.reshape(128, IN, H).transpose(0, 2, 1).reshape(128, H * IN)
    b1p = be1b.reshape(IN, H).transpose(1, 0).reshape(1, H * IN)
    W2p = jnp.zeros((256, 128), jnp.float32)
    W2p = W2p.at[:, :H * H].set(
        We2b.reshape(256, H, H).transpose(0, 2, 1).reshape(256, H * H))
    b2p = jnp.zeros((1, 128), jnp.float32)
    b2p = b2p.at[:, :H * H].set(be2b.reshape(H, H).transpose(1, 0).reshape(1, H * H))
    j1 = jnp.arange(H * IN)[:, None]
    sel1 = ((j1 // IN == jnp.arange(128)[None, :])
            & (j1 // IN < H)).astype(jnp.float32)
    c8 = (jnp.arange(128)[None, :] == 8).astype(jnp.float32)
    j2 = jnp.arange(128)[:, None]
    sel2 = ((j2 // H == jnp.arange(128)[None, :])
            & (j2 < H * H)).astype(jnp.float32)
    ji = jnp.arange(128)[None, :]
    rep = ((jnp.arange(H)[:, None] == ji % H)
           & (ji < H * H)).astype(jnp.float32)

    gather = _sc_gather(jnp.float32)
    scat = _sc_scatter_add()

    xs = gather(src, x)

    msg1 = pl.pallas_call(
        _k1_body,
        grid=(E // _B1,),
        in_specs=[
            pl.BlockSpec((_B1, ED), lambda i: (i, 0)),
            pl.BlockSpec((_B1, IN), lambda i: (i, 0)),
            pl.BlockSpec((ED, 128), lambda i: (0, 0)),
            pl.BlockSpec((1, 128), lambda i: (0, 0)),
            pl.BlockSpec((128, H * IN), lambda i: (0, 0)),
            pl.BlockSpec((1, H * IN), lambda i: (0, 0)),
            pl.BlockSpec((H * IN, 128), lambda i: (0, 0)),
            pl.BlockSpec((1, 128), lambda i: (0, 0)),
        ],
        out_specs=pl.BlockSpec((_B1, 128), lambda i: (i, 0)),
        out_shape=jax.ShapeDtypeStruct((E, 128), jnp.float32),
    )(edge_attr, xs, We1a, be1a.reshape(1, 128),
      W1p.astype(jnp.bfloat16), b1p, sel1.astype(jnp.bfloat16), c8)

    s1p = scat(dst, msg1)

    h1t, cnt8 = pl.pallas_call(
        _k2_body,
        in_specs=[_full((2 * _TR, 128)), _full((N, IN)), _full((IN, H)),
                  _full((1, H)), _full((1, H)), _full((1, H)), _full((H, 128))],
        out_specs=[_full((N, 128)), _full((N, 8))],
        out_shape=[jax.ShapeDtypeStruct((N, 128), jnp.float32),
                   jax.ShapeDtypeStruct((N, 8), jnp.float32)],
    )(s1p, x, Wr1, b1.reshape(1, H), g1.reshape(1, H), bt1.reshape(1, H), rep)

    hs = gather(src, h1t)

    msg2 = pl.pallas_call(
        _k3_body,
        grid=(E // _B3,),
        in_specs=[
            pl.BlockSpec((_B3, ED), lambda i: (i, 0)),
            pl.BlockSpec((_B3, 128), lambda i: (i, 0)),
            pl.BlockSpec((ED, 256), lambda i: (0, 0)),
            pl.BlockSpec((1, 256), lambda i: (0, 0)),
            pl.BlockSpec((256, 128), lambda i: (0, 0)),
            pl.BlockSpec((1, 128), lambda i: (0, 0)),
            pl.BlockSpec((128, 128), lambda i: (0, 0)),
        ],
        out_specs=pl.BlockSpec((_B3, 128), lambda i: (i, 0)),
        out_shape=jax.ShapeDtypeStruct((E, 128), jnp.float32),
    )(edge_attr, hs, We2a, be2a.reshape(1, 256),
      W2p.astype(jnp.bfloat16), b2p, sel2.astype(jnp.bfloat16))

    s2p = scat(dst, msg2)

    out = pl.pallas_call(
        _k4_body,
        in_specs=[_full((2 * _TR, 128)), _full((N, 128)), _full((N, 8)),
                  _full((1, N)), _full((H, H)), _full((1, H)), _full((1, H)),
                  _full((1, H)), _full((2 * H, H)), _full((1, H)),
                  _full((H, H // 2)), _full((1, H // 2)),
                  _full((H // 2, 1)), _full((1, 1))],
        out_specs=_full((NG, 1)),
        out_shape=jax.ShapeDtypeStruct((NG, 1), jnp.float32),
    )(s2p, h1t, cnt8, batch.reshape(1, N), Wr2, b2.reshape(1, H),
      g2.reshape(1, H), bt2.reshape(1, H), Wf1, bf1.reshape(1, H),
      Wf2, bf2.reshape(1, H // 2), Wf3, bf3.reshape(1, 1))

    return out


# speedup vs baseline: 1.1555x; 1.0300x over previous
import functools

import jax
import jax.numpy as jnp
from jax import lax
from jax.experimental import pallas as pl
from jax.experimental.pallas import tpu as pltpu
from jax.experimental.pallas import tpu_sc as plsc

N = 10000
E = 160000
IN = 128
ED = 16
H = 8
NG = 100

_NC = 2
_NS = 16
_NW = _NC * _NS
_CH = 128
_NCHK = E // _CH
_GI = 39
_SI = 78
_NP = 10240
_NPH = _NP // 2
_TR = _NPH + 128
_RPC = _TR // _NS


def _sc_mesh():
    return plsc.VectorSubcoreMesh(core_axis_name="c", subcore_axis_name="s")


def _sc_gather(dtype):
    @functools.partial(
        pl.kernel,
        mesh=_sc_mesh(),
        out_type=jax.ShapeDtypeStruct((E, 128), dtype),
        scratch_types=[
            pltpu.VMEM((_CH,), jnp.int32),
            pltpu.VMEM((_CH,), jnp.int32),
            pltpu.VMEM((_CH, 128), dtype),
            pltpu.VMEM((_CH, 128), dtype),
            pltpu.SemaphoreType.DMA,
            pltpu.SemaphoreType.DMA,
        ],
    )
    def g(idx_hbm, table_hbm, out_hbm, idxA, idxB, rowA, rowB, semA, semB):
        w = lax.axis_index("s") * _NC + lax.axis_index("c")

        def fetch(c, idxv, rowv, sem):
            pltpu.sync_copy(idx_hbm.at[pl.ds(pl.multiple_of(c * _CH, 8), _CH)],
                            idxv)
            pltpu.async_copy(table_hbm.at[idxv], rowv, sem)

        def drain(idxv, rowv, sem):
            pltpu.make_async_copy(table_hbm.at[idxv], rowv, sem).wait()

        def store(c, rowv):
            pltpu.sync_copy(
                rowv, out_hbm.at[pl.ds(pl.multiple_of(c * _CH, 8), _CH)])

        fetch(w, idxA, rowA, semA)

        def pair(j, carry):
            cA = w + 32 * (2 * j)
            fetch(cA + 32, idxB, rowB, semB)
            drain(idxA, rowA, semA)
            store(cA, rowA)

            @pl.when(2 * j + 2 < _GI)
            def _():
                fetch(cA + 64, idxA, rowA, semA)

            drain(idxB, rowB, semB)
            store(cA + 32, rowB)
            return carry

        lax.fori_loop(0, _GI // 2, pair, 0)
        cL = w + 32 * (_GI - 1)
        drain(idxA, rowA, semA)
        store(cL, rowA)

        @pl.when(w < 2)
        def _():
            cT = _NW * _GI + w
            fetch(cT, idxA, rowA, semA)
            drain(idxA, rowA, semA)
            store(cT, rowA)

    return g


def _sc_scatter_add():
    @functools.partial(
        pl.kernel,
        mesh=_sc_mesh(),
        out_type=jax.ShapeDtypeStruct((2 * _TR, 128), jnp.float32),
        scratch_types=[
            pltpu.VMEM((_CH,), jnp.int32),
            pltpu.VMEM((_CH,), jnp.int32),
            pltpu.VMEM((_CH, 128), jnp.float32),
            pltpu.VMEM((_CH, 128), jnp.float32),
            pltpu.VMEM((_RPC, 128), jnp.float32),
            pltpu.VMEM_SHARED((_TR, 128), jnp.float32),
            pltpu.SemaphoreType.DMA,
            pltpu.SemaphoreType.DMA,
        ],
    )
    def s(idx_hbm, msg_hbm, out_hbm, idxA, idxB, msgA, msgB, tmp_v, acc,
          semA, semB):
        cid = lax.axis_index("c")
        sid = lax.axis_index("s")
        lo = cid * _NPH

        def zbody(i, carry):
            for j in range(8):
                tmp_v[i, pl.ds(j * 16, 16)] = jnp.zeros((16,), jnp.float32)
            return carry

        lax.fori_loop(0, _RPC, zbody, 0)
        pltpu.sync_copy(tmp_v, acc.at[pl.ds(sid * _RPC, _RPC)])
        plsc.subcore_barrier()

        def fetch(c, idxv, msgv, sem):
            pltpu.sync_copy(idx_hbm.at[pl.ds(pl.multiple_of(c * _CH, 8), _CH)],
                            idxv)
            pltpu.async_copy(
                msg_hbm.at[pl.ds(pl.multiple_of(c * _CH, 8), _CH)], msgv, sem)
            for k in range(8):
                v = idxv[pl.ds(k * 16, 16)] - lo
                ok = (v >= 0) & (v < _NPH)
                idxv[pl.ds(k * 16, 16)] = jnp.where(ok, v, _NPH)

        def accum(c, idxv, msgv, sem):
            pltpu.make_async_copy(
                msg_hbm.at[pl.ds(pl.multiple_of(c * _CH, 8), _CH)], msgv,
                sem).wait()
            pltpu.sync_copy(msgv, acc.at[idxv], add=True)

        fetch(sid, idxA, msgA, semA)

        def pair(j, carry):
            cA = sid + 16 * (2 * j)
            fetch(cA + 16, idxB, msgB, semB)
            accum(cA, idxA, msgA, semA)

            @pl.when(2 * j + 2 < _SI)
            def _():
                fetch(cA + 32, idxA, msgA, semA)

            accum(cA + 16, idxB, msgB, semB)
            return carry

        lax.fori_loop(0, _SI // 2, pair, 0)

        @pl.when(sid < 2)
        def _():
            cT = _NS * _SI + sid
            fetch(cT, idxA, msgA, semA)
            accum(cT, idxA, msgA, semA)

        plsc.subcore_barrier()
        pltpu.sync_copy(acc.at[pl.ds(sid * _RPC, _RPC)], tmp_v)
        pltpu.sync_copy(
            tmp_v,
            out_hbm.at[pl.ds(pl.multiple_of(cid * _TR + sid * _RPC, 8), _RPC)])

    return s


_B1 = 4000
_B3 = 4000


def _bdot(a, b):
    return jnp.dot(a.astype(jnp.bfloat16), b,
                   preferred_element_type=jnp.float32)


def _k1_body(ea_ref, xs_ref, Wa_ref, ba_ref, Wp_ref, bp_ref, sel_ref, c8_ref,
             out_ref):
    eh = jnp.maximum(ea_ref[...] @ Wa_ref[...] + ba_ref[...], 0.0)
    W = _bdot(eh, Wp_ref[...]) + bp_ref[...]
    xst = jnp.concatenate([xs_ref[...]] * H, axis=1)
    out_ref[...] = _bdot(W * xst, sel_ref[...]) + c8_ref[...]


def _k3_body(ea_ref, hs_ref, Wa_ref, ba_ref, Wp_ref, bp_ref, sel_ref, out_ref):
    eh = jnp.maximum(ea_ref[...] @ Wa_ref[...] + ba_ref[...], 0.0)
    W = _bdot(eh, Wp_ref[...]) + bp_ref[...]
    out_ref[...] = _bdot(W * hs_ref[...], sel_ref[...])


def _seg(s_ref):
    return jnp.concatenate(
        [s_ref[:_NPH], s_ref[_TR:_TR + (N - _NPH)]], axis=0)


def _k2_body(s1p_ref, x_ref, Wr_ref, b_ref, g_ref, bt_ref, rep_ref,
             h1t_ref, cnt_ref):
    s = _seg(s1p_ref)
    cnt = jnp.maximum(s[:, 8:9], 1.0)
    pre = s[:, :H] / cnt + x_ref[...] @ Wr_ref[...] + b_ref[...]
    m = jnp.mean(pre, axis=0, keepdims=True)
    v = jnp.mean((pre - m) ** 2, axis=0, keepdims=True)
    h = jnp.maximum((pre - m) / jnp.sqrt(v + 1e-5) * g_ref[...] + bt_ref[...],
                    0.0)
    h1t_ref[...] = h @ rep_ref[...]
    cnt_ref[...] = jnp.broadcast_to(cnt, (N, 8))


_NGP = 104


def _k4_body(s2p_ref, h1t_ref, cnt_ref, batch_ref, Wr_ref, b_ref, g_ref,
             bt_ref, Wf1_ref, bf1_ref, Wf2_ref, bf2_ref, Wf3_ref, bf3_ref,
             out_ref):
    s = _seg(s2p_ref)
    cnt = jnp.maximum(cnt_ref[:, :1], 1.0)
    h1 = h1t_ref[:, :H]
    pre = s[:, :H] / cnt + h1 @ Wr_ref[...] + b_ref[...]
    m = jnp.mean(pre, axis=0, keepdims=True)
    v = jnp.mean((pre - m) ** 2, axis=0, keepdims=True)
    h2 = jnp.maximum((pre - m) / jnp.sqrt(v + 1e-5) * g_ref[...] + bt_ref[...],
                     0.0)

    row = lax.broadcasted_iota(jnp.int32, (_NGP, N), 0)
    ohT = (row == batch_ref[...]).astype(jnp.float32)
    gs = jnp.dot(ohT, h2)
    cg = jnp.maximum(jnp.sum(ohT, axis=1, keepdims=True), 1.0)
    z = jnp.concatenate([gs / cg, gs], axis=1)
    z = jnp.maximum(z @ Wf1_ref[...] + bf1_ref[...], 0.0)
    z = jnp.maximum(z @ Wf2_ref[...] + bf2_ref[...], 0.0)
    out_ref[...] = (z @ Wf3_ref[...] + bf3_ref[...])[:NG]


def _full(shape):
    return pl.BlockSpec(shape, lambda: tuple(0 for _ in shape))


def kernel(x, edge_index, edge_attr, batch,
           We1a, be1a, We1b, be1b, Wr1, b1, g1, bt1,
           We2a, be2a, We2b, be2b, Wr2, b2, g2, bt2,
           Wf1, bf1, Wf2, bf2, Wf3, bf3):
    src = edge_index[0]
    dst = edge_index[1]

    W1p = We1b.reshape(128, IN, H).transpose(0, 2, 1).reshape(128, H * IN)
    b1p = be1b.reshape(IN, H).transpose(1, 0).reshape(1, H * IN)
    W2p = jnp.zeros((256, 128), jnp.float32)
    W2p = W2p.at[:, :H * H].set(
        We2b.reshape(256, H, H).transpose(0, 2, 1).reshape(256, H * H))
    b2p = jnp.zeros((1, 128), jnp.float32)
    b2p = b2p.at[:, :H * H].set(be2b.reshape(H, H).transpose(1, 0).reshape(1, H * H))
    j1 = jnp.arange(H * IN)[:, None]
    sel1 = ((j1 // IN == jnp.arange(128)[None, :])
            & (j1 // IN < H)).astype(jnp.float32)
    c8 = (jnp.arange(128)[None, :] == 8).astype(jnp.float32)
    j2 = jnp.arange(128)[:, None]
    sel2 = ((j2 // H == jnp.arange(128)[None, :])
            & (j2 < H * H)).astype(jnp.float32)
    ji = jnp.arange(128)[None, :]
    rep = ((jnp.arange(H)[:, None] == ji % H)
           & (ji < H * H)).astype(jnp.float32)

    gather = _sc_gather(jnp.float32)
    scat = _sc_scatter_add()

    xs = gather(src, x)

    msg1 = pl.pallas_call(
        _k1_body,
        grid=(E // _B1,),
        in_specs=[
            pl.BlockSpec((_B1, ED), lambda i: (i, 0)),
            pl.BlockSpec((_B1, IN), lambda i: (i, 0)),
            pl.BlockSpec((ED, 128), lambda i: (0, 0)),
            pl.BlockSpec((1, 128), lambda i: (0, 0)),
            pl.BlockSpec((128, H * IN), lambda i: (0, 0)),
            pl.BlockSpec((1, H * IN), lambda i: (0, 0)),
            pl.BlockSpec((H * IN, 128), lambda i: (0, 0)),
            pl.BlockSpec((1, 128), lambda i: (0, 0)),
        ],
        out_specs=pl.BlockSpec((_B1, 128), lambda i: (i, 0)),
        out_shape=jax.ShapeDtypeStruct((E, 128), jnp.float32),
    )(edge_attr, xs, We1a, be1a.reshape(1, 128),
      W1p.astype(jnp.bfloat16), b1p, sel1.astype(jnp.bfloat16), c8)

    s1p = scat(dst, msg1)

    h1t, cnt8 = pl.pallas_call(
        _k2_body,
        in_specs=[_full((2 * _TR, 128)), _full((N, IN)), _full((IN, H)),
                  _full((1, H)), _full((1, H)), _full((1, H)), _full((H, 128))],
        out_specs=[_full((N, 128)), _full((N, 8))],
        out_shape=[jax.ShapeDtypeStruct((N, 128), jnp.float32),
                   jax.ShapeDtypeStruct((N, 8), jnp.float32)],
    )(s1p, x, Wr1, b1.reshape(1, H), g1.reshape(1, H), bt1.reshape(1, H), rep)

    hs = gather(src, h1t)

    msg2 = pl.pallas_call(
        _k3_body,
        grid=(E // _B3,),
        in_specs=[
            pl.BlockSpec((_B3, ED), lambda i: (i, 0)),
            pl.BlockSpec((_B3, 128), lambda i: (i, 0)),
            pl.BlockSpec((ED, 256), lambda i: (0, 0)),
            pl.BlockSpec((1, 256), lambda i: (0, 0)),
            pl.BlockSpec((256, 128), lambda i: (0, 0)),
            pl.BlockSpec((1, 128), lambda i: (0, 0)),
            pl.BlockSpec((128, 128), lambda i: (0, 0)),
        ],
        out_specs=pl.BlockSpec((_B3, 128), lambda i: (i, 0)),
        out_shape=jax.ShapeDtypeStruct((E, 128), jnp.float32),
    )(edge_attr, hs, We2a, be2a.reshape(1, 256),
      W2p.astype(jnp.bfloat16), b2p, sel2.astype(jnp.bfloat16))

    s2p = scat(dst, msg2)

    out = pl.pallas_call(
        _k4_body,
        in_specs=[_full((2 * _TR, 128)), _full((N, 128)), _full((N, 8)),
                  _full((1, N)), _full((H, H)), _full((1, H)), _full((1, H)),
                  _full((1, H)), _full((2 * H, H)), _full((1, H)),
                  _full((H, H // 2)), _full((1, H // 2)),
                  _full((H // 2, 1)), _full((1, 1))],
        out_specs=_full((NG, 1)),
        out_shape=jax.ShapeDtypeStruct((NG, 1), jnp.float32),
    )(s2p, h1t, cnt8, batch.reshape(1, N), Wr2, b2.reshape(1, H),
      g2.reshape(1, H), bt2.reshape(1, H), Wf1, bf1.reshape(1, H),
      Wf2, bf2.reshape(1, H // 2), Wf3, bf3.reshape(1, 1))

    return out
